# initial kernel scaffold (unmeasured)
import jax
import jax.numpy as jnp
from jax import lax
from jax.experimental import pallas as pl
from jax.experimental.pallas import tpu as pltpu

N_ROWS = 512
N_COLS = 256


def kernel(x, dest):
    dest_row = dest.reshape(1, N_ROWS)

    def body(x_ref, d_ref, out_ref, stage_ref, comm_ref, send_sem, recv_sem):
        my_x = lax.axis_index("x")
        my_y = lax.axis_index("y")
        my_z = lax.axis_index("z")
        nbr = (my_x, 1 - my_y, my_z)

        barrier = pltpu.get_barrier_semaphore()
        pl.semaphore_signal(
            barrier, inc=1, device_id=nbr, device_id_type=pl.DeviceIdType.MESH
        )
        pl.semaphore_wait(barrier, 1)

        d = d_ref[...]
        m0 = d == 0
        m0f = m0.astype(jnp.float32)

        r0 = lax.broadcasted_iota(jnp.int32, (N_ROWS, N_ROWS), 0)
        r1 = lax.broadcasted_iota(jnp.int32, (N_ROWS, N_ROWS), 1)

        tri = (r0 <= r1).astype(jnp.float32)
        c0 = jnp.dot(m0f, tri, preferred_element_type=jnp.float32)
        pos = lax.broadcasted_iota(jnp.int32, (1, N_ROWS), 1).astype(jnp.float32)
        c1 = (pos + 1.0) - c0
        n0 = jnp.sum(m0.astype(jnp.int32))
        n0f = n0.astype(jnp.float32)

        p = jnp.where(m0, c0 - 1.0, c1 - 1.0 + n0f)
        p_i = p.astype(jnp.int32)

        perm = (p_i == r0).astype(jnp.bfloat16)
        xb = x_ref[...].astype(jnp.bfloat16)
        sorted_x = jnp.dot(perm, xb, preferred_element_type=jnp.bfloat16)
        stage_ref[...] = sorted_x

        rdma = pltpu.make_async_remote_copy(
            src_ref=stage_ref,
            dst_ref=comm_ref,
            send_sem=send_sem,
            recv_sem=recv_sem,
            device_id=nbr,
            device_id_type=pl.DeviceIdType.MESH,
        )
        rdma.start()
        rdma.wait()

        n0m = lax.rem(n0, N_ROWS)
        shift_ok = lax.rem(r0 - r1 + 2 * N_ROWS, N_ROWS) == n0m
        lt = r0 < n0
        nbr_region = jnp.logical_xor(lt, my_y == 0)
        merge = jnp.logical_and(shift_ok, nbr_region).astype(jnp.bfloat16)
        nbr_part = jnp.dot(merge, comm_ref[...], preferred_element_type=jnp.bfloat16)

        rc = lax.broadcasted_iota(jnp.int32, (N_ROWS, 1), 0)
        nbr_rows = jnp.logical_xor(rc < n0, my_y == 0)
        out_ref[...] = jnp.where(nbr_rows, nbr_part, sorted_x).astype(jnp.float32)

    return pl.pallas_call(
        body,
        out_shape=jax.ShapeDtypeStruct((N_ROWS, N_COLS), jnp.float32),
        in_specs=[
            pl.BlockSpec(memory_space=pltpu.VMEM),
            pl.BlockSpec(memory_space=pltpu.VMEM),
        ],
        out_specs=pl.BlockSpec(memory_space=pltpu.VMEM),
        scratch_shapes=[
            pltpu.VMEM((N_ROWS, N_COLS), jnp.bfloat16),
            pltpu.VMEM((N_ROWS, N_COLS), jnp.bfloat16),
            pltpu.SemaphoreType.DMA,
            pltpu.SemaphoreType.DMA,
        ],
        compiler_params=pltpu.CompilerParams(collective_id=0),
    )(x, dest_row)


# baseline (device time: 9773 ns/iter reference)
import jax
import jax.numpy as jnp
from jax import lax
from jax.experimental import pallas as pl
from jax.experimental.pallas import tpu as pltpu

N_ROWS = 512
N_COLS = 256


def kernel(x, dest):
    dest_row = dest.reshape(1, N_ROWS)

    def body(x_ref, d_ref, out_ref, stage_ref, comm_ref, send_sem, recv_sem):
        my_x = lax.axis_index("x")
        my_y = lax.axis_index("y")
        my_z = lax.axis_index("z")
        nbr = (my_x, 1 - my_y, my_z)

        barrier = pltpu.get_barrier_semaphore()
        pl.semaphore_signal(
            barrier, inc=1, device_id=nbr, device_id_type=pl.DeviceIdType.MESH
        )
        pl.semaphore_wait(barrier, 1)

        d = d_ref[...]
        m0 = d == 0
        m0f = m0.astype(jnp.float32)

        r0 = lax.broadcasted_iota(jnp.int32, (N_ROWS, N_ROWS), 0)
        r1 = lax.broadcasted_iota(jnp.int32, (N_ROWS, N_ROWS), 1)

        tri = (r0 <= r1).astype(jnp.float32)
        c0 = jnp.dot(m0f, tri, preferred_element_type=jnp.float32)
        pos = lax.broadcasted_iota(jnp.int32, (1, N_ROWS), 1).astype(jnp.float32)
        c1 = (pos + 1.0) - c0
        n0 = jnp.sum(m0.astype(jnp.int32))
        n0f = n0.astype(jnp.float32)

        p = jnp.where(m0, c0 - 1.0, c1 - 1.0 + n0f)
        p_i = p.astype(jnp.int32)

        perm = (p_i == r0).astype(jnp.bfloat16)
        xb = x_ref[...].astype(jnp.bfloat16)
        sorted_x = jnp.dot(
            perm, xb, preferred_element_type=jnp.float32
        ).astype(jnp.bfloat16)
        stage_ref[...] = sorted_x

        rdma = pltpu.make_async_remote_copy(
            src_ref=stage_ref,
            dst_ref=comm_ref,
            send_sem=send_sem,
            recv_sem=recv_sem,
            device_id=nbr,
            device_id_type=pl.DeviceIdType.MESH,
        )
        rdma.start()
        rdma.wait()

        n0m = lax.rem(n0, N_ROWS)
        shift_ok = lax.rem(r0 - r1 + 2 * N_ROWS, N_ROWS) == n0m
        lt = r0 < n0
        nbr_region = jnp.logical_xor(lt, my_y == 0)
        merge = jnp.logical_and(shift_ok, nbr_region).astype(jnp.bfloat16)
        nbr_part = jnp.dot(
            merge, comm_ref[...], preferred_element_type=jnp.float32
        ).astype(jnp.bfloat16)

        rc = lax.broadcasted_iota(jnp.int32, (N_ROWS, 1), 0)
        nbr_rows = jnp.logical_xor(rc < n0, my_y == 0)
        out_ref[...] = jnp.where(nbr_rows, nbr_part, sorted_x).astype(jnp.float32)

    return pl.pallas_call(
        body,
        out_shape=jax.ShapeDtypeStruct((N_ROWS, N_COLS), jnp.float32),
        in_specs=[
            pl.BlockSpec(memory_space=pltpu.VMEM),
            pl.BlockSpec(memory_space=pltpu.VMEM),
        ],
        out_specs=pl.BlockSpec(memory_space=pltpu.VMEM),
        scratch_shapes=[
            pltpu.VMEM((N_ROWS, N_COLS), jnp.bfloat16),
            pltpu.VMEM((N_ROWS, N_COLS), jnp.bfloat16),
            pltpu.SemaphoreType.DMA,
            pltpu.SemaphoreType.DMA,
        ],
        compiler_params=pltpu.CompilerParams(collective_id=0),
    )(x, dest_row)


# device time: 8399 ns/iter; 1.1636x vs baseline; 1.1636x over previous
import jax
import jax.numpy as jnp
from jax import lax
from jax.experimental import pallas as pl
from jax.experimental.pallas import tpu as pltpu

N_ROWS = 512
N_COLS = 256
CHUNK = 32
N_CHUNKS = N_ROWS // CHUNK


def kernel(x, dest):
    dest_row = dest.reshape(1, N_ROWS)

    def body(x_ref, d_ref, out_ref, stage_ref, comm_ref, send_sems, recv_sems):
        my_x = lax.axis_index("x")
        my_y = lax.axis_index("y")
        my_z = lax.axis_index("z")
        nbr = (my_x, 1 - my_y, my_z)
        i_am_y0 = my_y == 0

        barrier = pltpu.get_barrier_semaphore()
        pl.semaphore_signal(
            barrier, inc=1, device_id=nbr, device_id_type=pl.DeviceIdType.MESH
        )

        d = d_ref[...]
        m0 = d == 0
        m0f = m0.astype(jnp.float32)

        r0 = lax.broadcasted_iota(jnp.int32, (N_ROWS, N_ROWS), 0)
        r1 = lax.broadcasted_iota(jnp.int32, (N_ROWS, N_ROWS), 1)

        tri = (r0 <= r1).astype(jnp.bfloat16)
        c0 = jnp.dot(
            m0f.astype(jnp.bfloat16), tri, preferred_element_type=jnp.float32
        )
        pos = lax.broadcasted_iota(jnp.int32, (1, N_ROWS), 1).astype(jnp.float32)
        c1 = (pos + 1.0) - c0
        n0 = jnp.sum(m0.astype(jnp.int32))
        nbr_n0 = N_ROWS - n0
        n0f = n0.astype(jnp.float32)

        p = jnp.where(m0, c0 - 1.0, c1 - 1.0 + n0f)
        p_i = p.astype(jnp.int32)

        perm = (p_i == r0).astype(jnp.bfloat16)
        xb = x_ref[...].astype(jnp.bfloat16)
        sorted_x = jnp.dot(
            perm, xb, preferred_element_type=jnp.float32
        ).astype(jnp.bfloat16)
        stage_ref[...] = sorted_x

        pl.semaphore_wait(barrier, 1)
        rdmas = []
        for k in range(N_CHUNKS):
            sl = pl.ds(k * CHUNK, CHUNK)
            rdma = pltpu.make_async_remote_copy(
                src_ref=stage_ref.at[sl],
                dst_ref=comm_ref.at[sl],
                send_sem=send_sems.at[k],
                recv_sem=recv_sems.at[k],
                device_id=nbr,
                device_id_type=pl.DeviceIdType.MESH,
            )
            rdmas.append(rdma)
            send_k = jnp.where(i_am_y0, (k + 1) * CHUNK > n0, k * CHUNK < n0)

            @pl.when(send_k)
            def _(rdma=rdma):
                rdma.start()

        rc = lax.broadcasted_iota(jnp.int32, (N_ROWS, 1), 0)
        nbr_rows = jnp.logical_xor(rc < n0, i_am_y0)

        for k in range(N_CHUNKS):
            expect_k = jnp.where(
                i_am_y0, k * CHUNK < nbr_n0, (k + 1) * CHUNK > nbr_n0
            )

            @pl.when(expect_k)
            def _(rdma=rdmas[k]):
                rdma.wait_recv()

        rolled = pltpu.roll(comm_ref[...], n0, 0)
        out_ref[...] = jnp.where(nbr_rows, rolled, sorted_x).astype(jnp.float32)

        for k in range(N_CHUNKS):
            send_k = jnp.where(i_am_y0, (k + 1) * CHUNK > n0, k * CHUNK < n0)

            @pl.when(send_k)
            def _(rdma=rdmas[k]):
                rdma.wait_send()

    return pl.pallas_call(
        body,
        out_shape=jax.ShapeDtypeStruct((N_ROWS, N_COLS), jnp.float32),
        in_specs=[
            pl.BlockSpec(memory_space=pltpu.VMEM),
            pl.BlockSpec(memory_space=pltpu.VMEM),
        ],
        out_specs=pl.BlockSpec(memory_space=pltpu.VMEM),
        scratch_shapes=[
            pltpu.VMEM((N_ROWS, N_COLS), jnp.bfloat16),
            pltpu.VMEM((N_ROWS, N_COLS), jnp.bfloat16),
            pltpu.SemaphoreType.DMA((N_CHUNKS,)),
            pltpu.SemaphoreType.DMA((N_CHUNKS,)),
        ],
        compiler_params=pltpu.CompilerParams(collective_id=0),
    )(x, dest_row)


# device time: 8310 ns/iter; 1.1761x vs baseline; 1.0107x over previous
import contextlib
import os

import jax
import jax.numpy as jnp
from jax import lax
from jax.experimental import pallas as pl
from jax.experimental.pallas import tpu as pltpu

N_ROWS = 512
N_COLS = 256
CHUNK = int(os.environ.get("KERNEL_CHUNK", "128"))
N_CHUNKS = N_ROWS // CHUNK

_PROFILE = os.environ.get("KERNEL_SCOPES") == "1"
_NOCOMM = os.environ.get("KERNEL_NOCOMM") == "1"


def _scope(name):
    return jax.named_scope(name) if _PROFILE else contextlib.nullcontext()


def kernel(x, dest):
    dest_row = dest.reshape(1, N_ROWS)

    def body(
        x_hbm,
        d_hbm,
        out_hbm,
        x_vmem,
        d_vmem,
        out_vmem,
        stage_ref,
        comm_ref,
        send_sems,
        recv_sems,
        in_sems,
        out_sem,
    ):
        my_x = lax.axis_index("x")
        my_y = lax.axis_index("y")
        my_z = lax.axis_index("z")
        nbr = (my_x, 1 - my_y, my_z)
        i_am_y0 = my_y == 0

        barrier = pltpu.get_barrier_semaphore()
        pl.semaphore_signal(
            barrier, inc=1, device_id=nbr, device_id_type=pl.DeviceIdType.MESH
        )

        copy_d = pltpu.make_async_copy(d_hbm, d_vmem, in_sems.at[0])
        copy_d.start()
        copy_x = pltpu.make_async_copy(x_hbm, x_vmem, in_sems.at[1])
        copy_x.start()

        with _scope("prep"):
            copy_d.wait()
            d = d_vmem[...]
            m0 = d == 0

            r0 = lax.broadcasted_iota(jnp.int32, (N_ROWS, N_ROWS), 0)
            r1 = lax.broadcasted_iota(jnp.int32, (N_ROWS, N_ROWS), 1)

            tri = (r0 <= r1).astype(jnp.bfloat16)
            c0 = jnp.dot(
                m0.astype(jnp.bfloat16), tri, preferred_element_type=jnp.float32
            )
            pos = lax.broadcasted_iota(jnp.int32, (1, N_ROWS), 1).astype(
                jnp.float32
            )
            c1 = (pos + 1.0) - c0
            n0 = jnp.sum(m0.astype(jnp.int32))
            nbr_n0 = N_ROWS - n0
            n0f = n0.astype(jnp.float32)

            p = jnp.where(m0, c0 - 1.0, c1 - 1.0 + n0f)
            p_i = p.astype(jnp.int32)

        with _scope("perm_mm"):
            perm = (p_i == r0).astype(jnp.bfloat16)
            copy_x.wait()
            xb = x_vmem[...].astype(jnp.bfloat16)
            sorted_x = jnp.dot(
                perm, xb, preferred_element_type=jnp.float32
            ).astype(jnp.bfloat16)
            stage_ref[...] = sorted_x

        if _NOCOMM:
            rc = lax.broadcasted_iota(jnp.int32, (N_ROWS, 1), 0)
            nbr_rows = jnp.logical_xor(rc < n0, i_am_y0)
            rolled = pltpu.roll(stage_ref[...], n0, 0)
            out_vmem[...] = jnp.where(nbr_rows, rolled, sorted_x).astype(
                jnp.float32
            )
            copy_out = pltpu.make_async_copy(out_vmem, out_hbm, out_sem)
            copy_out.start()
            copy_out.wait()
            return

        with _scope("barrier_wait"):
            pl.semaphore_wait(barrier, 1)
        rdmas = []
        with _scope("send"):
            for k in range(N_CHUNKS):
                sl = pl.ds(k * CHUNK, CHUNK)
                rdma = pltpu.make_async_remote_copy(
                    src_ref=stage_ref.at[sl],
                    dst_ref=comm_ref.at[sl],
                    send_sem=send_sems.at[k],
                    recv_sem=recv_sems.at[k],
                    device_id=nbr,
                    device_id_type=pl.DeviceIdType.MESH,
                )
                rdmas.append(rdma)
                send_k = jnp.where(i_am_y0, (k + 1) * CHUNK > n0, k * CHUNK < n0)

                @pl.when(send_k)
                def _(rdma=rdma):
                    rdma.start()

        rc = lax.broadcasted_iota(jnp.int32, (N_ROWS, 1), 0)
        nbr_rows = jnp.logical_xor(rc < n0, i_am_y0)

        with _scope("wait_recv"):
            for k in range(N_CHUNKS):
                expect_k = jnp.where(
                    i_am_y0, k * CHUNK < nbr_n0, (k + 1) * CHUNK > nbr_n0
                )

                @pl.when(expect_k)
                def _(rdma=rdmas[k]):
                    rdma.wait_recv()

        with _scope("merge"):
            rolled = pltpu.roll(comm_ref[...], n0, 0)
            out_vmem[...] = jnp.where(nbr_rows, rolled, sorted_x).astype(
                jnp.float32
            )
            copy_out = pltpu.make_async_copy(out_vmem, out_hbm, out_sem)
            copy_out.start()

        with _scope("wait_send"):
            for k in range(N_CHUNKS):
                send_k = jnp.where(i_am_y0, (k + 1) * CHUNK > n0, k * CHUNK < n0)

                @pl.when(send_k)
                def _(rdma=rdmas[k]):
                    rdma.wait_send()
            copy_out.wait()

    return pl.pallas_call(
        body,
        out_shape=jax.ShapeDtypeStruct((N_ROWS, N_COLS), jnp.float32),
        in_specs=[
            pl.BlockSpec(memory_space=pl.ANY),
            pl.BlockSpec(memory_space=pl.ANY),
        ],
        out_specs=pl.BlockSpec(memory_space=pl.ANY),
        scratch_shapes=[
            pltpu.VMEM((N_ROWS, N_COLS), jnp.float32),
            pltpu.VMEM((1, N_ROWS), jnp.int32),
            pltpu.VMEM((N_ROWS, N_COLS), jnp.float32),
            pltpu.VMEM((N_ROWS, N_COLS), jnp.bfloat16),
            pltpu.VMEM((N_ROWS, N_COLS), jnp.bfloat16),
            pltpu.SemaphoreType.DMA((N_CHUNKS,)),
            pltpu.SemaphoreType.DMA((N_CHUNKS,)),
            pltpu.SemaphoreType.DMA((2,)),
            pltpu.SemaphoreType.DMA,
        ],
        compiler_params=pltpu.CompilerParams(collective_id=0),
    )(x, dest_row)


# device time: 8064 ns/iter; 1.2119x vs baseline; 1.0305x over previous
import contextlib
import os

import jax
import jax.numpy as jnp
from jax import lax
from jax.experimental import pallas as pl
from jax.experimental.pallas import tpu as pltpu

N_ROWS = 512
N_COLS = 256
WIN = 128
N_WIN = N_ROWS // WIN
PAD = N_ROWS + WIN

_PROFILE = os.environ.get("KERNEL_SCOPES") == "1"
_NOCOMM = os.environ.get("KERNEL_NOCOMM") == "1"


def _scope(name):
    return jax.named_scope(name) if _PROFILE else contextlib.nullcontext()


def kernel(x, dest):
    dest_row = dest.reshape(1, N_ROWS)

    def body(x_ref, d_ref, out_ref, stage_ref, comm_ref, send_sems, recv_sems):
        my_x = lax.axis_index("x")
        my_y = lax.axis_index("y")
        my_z = lax.axis_index("z")
        nbr = (my_x, 1 - my_y, my_z)
        i_am_y0 = my_y == 0

        barrier = pltpu.get_barrier_semaphore()
        pl.semaphore_signal(
            barrier, inc=1, device_id=nbr, device_id_type=pl.DeviceIdType.MESH
        )

        with _scope("prep"):
            d = d_ref[...]
            m0 = d == 0

            r0 = lax.broadcasted_iota(jnp.int32, (N_ROWS, N_ROWS), 0)
            r1 = lax.broadcasted_iota(jnp.int32, (N_ROWS, N_ROWS), 1)

            tri = (r0 <= r1).astype(jnp.bfloat16)
            c0 = jnp.dot(
                m0.astype(jnp.bfloat16), tri, preferred_element_type=jnp.float32
            )
            pos = lax.broadcasted_iota(jnp.int32, (1, N_ROWS), 1).astype(
                jnp.float32
            )
            c1 = (pos + 1.0) - c0
            n0 = jnp.sum(m0.astype(jnp.int32))
            n0f = n0.astype(jnp.float32)

            p = jnp.where(m0, c0 - 1.0, c1 - 1.0 + n0f)
            p_i = p.astype(jnp.int32)

        with _scope("perm_mm"):
            perm = (p_i == r0).astype(jnp.bfloat16)
            xb = x_ref[...].astype(jnp.bfloat16)
            sorted_x = jnp.dot(
                perm, xb, preferred_element_type=jnp.float32
            ).astype(jnp.bfloat16)
            stage_ref[...] = sorted_x

        rc = lax.broadcasted_iota(jnp.int32, (N_ROWS, 1), 0)
        nbr_rows = jnp.logical_xor(rc < n0, i_am_y0)

        if _NOCOMM:
            rolled = pltpu.roll(stage_ref[...], n0, 0)
            out_ref[...] = jnp.where(nbr_rows, rolled, sorted_x)
            return

        nbr_n0 = N_ROWS - n0

        with _scope("barrier_wait"):
            pl.semaphore_wait(barrier, 1)
        rdmas = []
        with _scope("send"):
            for k in range(N_WIN):
                sl = pl.ds(k * WIN, WIN)
                rdma = pltpu.make_async_remote_copy(
                    src_ref=stage_ref.at[sl],
                    dst_ref=comm_ref.at[sl],
                    send_sem=send_sems.at[k],
                    recv_sem=recv_sems.at[k],
                    device_id=nbr,
                    device_id_type=pl.DeviceIdType.MESH,
                )
                rdmas.append(rdma)
                send_k = jnp.where(i_am_y0, (k + 1) * WIN > n0, k * WIN < n0)

                @pl.when(send_k)
                def _(rdma=rdma):
                    rdma.start()

        with _scope("wait_recv"):
            for k in range(N_WIN):
                expect_k = jnp.where(
                    i_am_y0, k * WIN < nbr_n0, (k + 1) * WIN > nbr_n0
                )

                @pl.when(expect_k)
                def _(rdma=rdmas[k]):
                    rdma.wait_recv()

        with _scope("merge"):
            rolled = pltpu.roll(comm_ref[...], n0, 0)
            out_ref[...] = jnp.where(nbr_rows, rolled, sorted_x)

        with _scope("wait_send"):
            for k in range(N_WIN):
                send_k = jnp.where(i_am_y0, (k + 1) * WIN > n0, k * WIN < n0)

                @pl.when(send_k)
                def _(rdma=rdmas[k]):
                    rdma.wait_send()

    return pl.pallas_call(
        body,
        out_shape=jax.ShapeDtypeStruct((N_ROWS, N_COLS), jnp.bfloat16),
        in_specs=[
            pl.BlockSpec(memory_space=pltpu.VMEM),
            pl.BlockSpec(memory_space=pltpu.VMEM),
        ],
        out_specs=pl.BlockSpec(memory_space=pltpu.VMEM),
        scratch_shapes=[
            pltpu.VMEM((N_ROWS, N_COLS), jnp.bfloat16),
            pltpu.VMEM((N_ROWS, N_COLS), jnp.bfloat16),
            pltpu.SemaphoreType.DMA((N_WIN,)),
            pltpu.SemaphoreType.DMA((N_WIN,)),
        ],
        compiler_params=pltpu.CompilerParams(collective_id=0),
    )(x, dest_row)


# device time: 8028 ns/iter; 1.2174x vs baseline; 1.0045x over previous
import contextlib
import os

import jax
import jax.numpy as jnp
from jax import lax
from jax.experimental import pallas as pl
from jax.experimental.pallas import tpu as pltpu

N_ROWS = 512
N_COLS = 256
WIN = 128
N_WIN = N_ROWS // WIN

_PROFILE = os.environ.get("KERNEL_SCOPES") == "1"
_NOCOMM = os.environ.get("KERNEL_NOCOMM") == "1"


def _scope(name):
    return jax.named_scope(name) if _PROFILE else contextlib.nullcontext()


def kernel(x, dest):
    dest_row = dest.reshape(1, N_ROWS)

    def body(x_ref, d_ref, out_ref, stage_ref, comm_ref, send_sems, recv_sems):
        my_x = lax.axis_index("x")
        my_y = lax.axis_index("y")
        my_z = lax.axis_index("z")
        nbr = (my_x, 1 - my_y, my_z)
        i_am_y0 = my_y == 0

        barrier = pltpu.get_barrier_semaphore()
        pl.semaphore_signal(
            barrier, inc=1, device_id=nbr, device_id_type=pl.DeviceIdType.MESH
        )

        with _scope("prep"):
            d = d_ref[...]
            m0 = d == 0

            r0 = lax.broadcasted_iota(jnp.int32, (N_ROWS, N_ROWS), 0)
            r1 = lax.broadcasted_iota(jnp.int32, (N_ROWS, N_ROWS), 1)

            tri = (r0 <= r1).astype(jnp.bfloat16)
            c0 = jnp.dot(
                m0.astype(jnp.bfloat16), tri, preferred_element_type=jnp.float32
            )
            pos = lax.broadcasted_iota(jnp.int32, (1, N_ROWS), 1).astype(
                jnp.float32
            )
            c1 = (pos + 1.0) - c0
            n0 = jnp.sum(m0.astype(jnp.int32))
            n0f = n0.astype(jnp.float32)

            p = jnp.where(m0, c0 - 1.0, c1 - 1.0 + n0f)
            p_i = p.astype(jnp.int32)

        with _scope("perm_mm"):
            perm = (p_i == r0).astype(jnp.bfloat16)
            xb = x_ref[...].astype(jnp.bfloat16)
            sorted_x = jnp.dot(
                perm, xb, preferred_element_type=jnp.float32
            ).astype(jnp.bfloat16)
            stage_ref[...] = sorted_x

        rc = lax.broadcasted_iota(jnp.int32, (N_ROWS, 1), 0)
        nbr_rows = jnp.logical_xor(rc < n0, i_am_y0)

        if _NOCOMM:
            rolled = pltpu.roll(stage_ref[...], n0, 0)
            out_ref[...] = jnp.where(nbr_rows, rolled, sorted_x)
            return

        nbr_n0 = N_ROWS - n0

        with _scope("barrier_wait"):
            pl.semaphore_wait(barrier, 1)
        rdmas = []
        with _scope("send"):
            for k in range(N_WIN):
                sl = pl.ds(k * WIN, WIN)
                rdma = pltpu.make_async_remote_copy(
                    src_ref=stage_ref.at[sl],
                    dst_ref=comm_ref.at[sl],
                    send_sem=send_sems.at[k],
                    recv_sem=recv_sems.at[k],
                    device_id=nbr,
                    device_id_type=pl.DeviceIdType.MESH,
                )
                rdmas.append(rdma)
                send_k = jnp.where(i_am_y0, (k + 1) * WIN > n0, k * WIN < n0)

                @pl.when(send_k)
                def _(rdma=rdma):
                    rdma.start()

        with _scope("wait_recv"):
            for k in range(N_WIN):
                expect_k = jnp.where(
                    i_am_y0, k * WIN < nbr_n0, (k + 1) * WIN > nbr_n0
                )

                @pl.when(expect_k)
                def _(rdma=rdmas[k]):
                    rdma.wait_recv()

        with _scope("merge"):
            rolled = pltpu.roll(comm_ref[...], n0, 0)
            out_ref[...] = jnp.where(nbr_rows, rolled, sorted_x)

        with _scope("wait_send"):
            for k in range(N_WIN):
                send_k = jnp.where(i_am_y0, (k + 1) * WIN > n0, k * WIN < n0)

                @pl.when(send_k)
                def _(rdma=rdmas[k]):
                    rdma.wait_send()

    return pl.pallas_call(
        body,
        out_shape=jax.ShapeDtypeStruct((N_ROWS, N_COLS), jnp.bfloat16),
        in_specs=[
            pl.BlockSpec(memory_space=pltpu.VMEM),
            pl.BlockSpec(memory_space=pltpu.VMEM),
        ],
        out_specs=pl.BlockSpec(memory_space=pltpu.VMEM),
        scratch_shapes=[
            pltpu.VMEM((N_ROWS, N_COLS), jnp.bfloat16),
            pltpu.VMEM((N_ROWS, N_COLS), jnp.bfloat16),
            pltpu.SemaphoreType.DMA((N_WIN,)),
            pltpu.SemaphoreType.DMA((N_WIN,)),
        ],
        compiler_params=pltpu.CompilerParams(collective_id=0),
    )(x, dest_row)
